# baseline (speedup 1.0000x reference)
import jax
import jax.numpy as jnp
from jax import lax
from jax.experimental import pallas as pl
from jax.experimental.pallas import tpu as pltpu
from jax.experimental.pallas import tpu_sc as plsc

NUM_CORES = 2
NUM_SUBCORES = 16
C = 128


def _split(r, o_ref):
    half = r.shape[1] // 2
    o_ref[0] = r[:, :half]
    o_ref[1] = r[:, half:]


def _mm_body(x_ref, w_ref, o_ref):
    _split(jnp.dot(x_ref[...], w_ref[...], preferred_element_type=jnp.float32),
           o_ref)


def _fuse_body(p_ref, w_ref, o_ref):
    h = jnp.maximum(jnp.concatenate([p_ref[0], p_ref[1]], axis=1), 0.0)
    _split(jnp.dot(h, w_ref[...], preferred_element_type=jnp.float32), o_ref)


def _make_sc_scatter(nh, acc_n, dh, chunks):
    rpt = acc_n // NUM_SUBCORES
    mesh = plsc.VectorSubcoreMesh(
        core_axis_name="c", subcore_axis_name="s",
        num_cores=NUM_CORES, num_subcores=NUM_SUBCORES)

    def body(h_hbm, src_hbm, dst_hbm, init_hbm, out_hbm,
             src_v, dst_v, rows0, rows1, acc, sem0, sem1):
        cid = lax.axis_index("c")
        sid = lax.axis_index("s")
        hc = h_hbm.at[cid]

        pltpu.sync_copy(src_hbm.at[sid], src_v)
        pltpu.sync_copy(dst_hbm.at[sid], dst_v)
        pltpu.sync_copy(init_hbm.at[cid].at[pl.ds(sid * rpt, rpt)],
                        acc.at[pl.ds(sid * rpt, rpt)])
        plsc.subcore_barrier()

        def gather(j, buf, sem):
            pltpu.async_copy(hc.at[src_v.at[j]], buf, sem)

        def gwait(j, buf, sem):
            pltpu.make_async_copy(hc.at[src_v.at[j]], buf, sem).wait()

        def scatter(j, buf):
            pltpu.sync_copy(buf, acc.at[dst_v.at[j]], add=True)

        gather(0, rows0, sem0)

        def pipe(j, k, fire_next=True):
            buf, sem = (rows0, sem0) if k % 2 == 0 else (rows1, sem1)
            nbuf, nsem = (rows1, sem1) if k % 2 == 0 else (rows0, sem0)
            if fire_next:
                gather(j + k + 1, nbuf, nsem)
            gwait(j + k, buf, sem)
            scatter(j + k, buf)

        def step(i, carry):
            j = 4 * i
            for k in range(4):
                pipe(j, k)
            return carry

        lax.fori_loop(0, chunks // 4 - 1, step, 0)
        j = chunks - 4
        for k in range(4):
            pipe(j, k, fire_next=k < 3)

        plsc.subcore_barrier()
        pltpu.sync_copy(acc.at[pl.ds(sid * rpt, rpt)],
                        out_hbm.at[cid].at[pl.ds(sid * rpt, rpt)])

    return pl.kernel(
        body,
        out_type=jax.ShapeDtypeStruct((NUM_CORES, acc_n, dh), jnp.float32),
        mesh=mesh,
        compiler_params=pltpu.CompilerParams(use_tc_tiling_on_sc=False),
        scratch_types=[
            pltpu.VMEM((chunks, C), jnp.int32),
            pltpu.VMEM((chunks, C), jnp.int32),
            pltpu.VMEM((C, dh), jnp.float32),
            pltpu.VMEM((C, dh), jnp.float32),
            pltpu.VMEM_SHARED((acc_n, dh), jnp.float32),
            pltpu.SemaphoreType.DMA,
            pltpu.SemaphoreType.DMA,
        ],
    )


def kernel(x, edge_index, W1, b1, W2, b2):
    n, d_in = x.shape
    d_hid = W1.shape[1]
    d_out = W2.shape[1]
    e = edge_index.shape[1]
    hh = d_hid // 2
    ho = d_out // 2

    chunks = (-(-e // (NUM_SUBCORES * C * 4))) * 4
    e_pad = NUM_SUBCORES * chunks * C
    acc_n = (-(-(n + 1) // (NUM_SUBCORES * 8))) * NUM_SUBCORES * 8
    dummy = n
    src = jnp.concatenate(
        [edge_index[0], jnp.zeros((e_pad - e,), jnp.int32)]
    ).reshape(NUM_SUBCORES, chunks, C)
    dst = jnp.concatenate(
        [edge_index[1], jnp.full((e_pad - e,), dummy, jnp.int32)]
    ).reshape(NUM_SUBCORES, chunks, C)
    init1 = jnp.broadcast_to(b1.reshape(2, 1, hh), (2, acc_n, hh))
    init2 = jnp.broadcast_to(b2.reshape(2, 1, ho), (2, acc_n, ho))

    h1 = pl.pallas_call(
        _mm_body,
        out_shape=jax.ShapeDtypeStruct((2, n, hh), jnp.float32))(x, W1)
    p1 = _make_sc_scatter(n, acc_n, hh, chunks)(h1, src, dst, init1)
    h2 = pl.pallas_call(
        _fuse_body,
        out_shape=jax.ShapeDtypeStruct((2, acc_n, ho), jnp.float32))(p1, W2)
    p2 = _make_sc_scatter(acc_n, acc_n, ho, chunks)(h2, src, dst, init2)
    return jnp.concatenate([p2[0, :n], p2[1, :n]], axis=1)


# speedup vs baseline: 1.3109x; 1.3109x over previous
import jax
import jax.numpy as jnp
from jax import lax
from jax.experimental import pallas as pl
from jax.experimental.pallas import tpu as pltpu
from jax.experimental.pallas import tpu_sc as plsc

NUM_CORES = 2
NUM_SUBCORES = 16
C = 128


def _split(r, o_ref):
    half = r.shape[1] // 2
    o_ref[0] = r[:, :half]
    o_ref[1] = r[:, half:]


def _mm_body(x_ref, w_ref, o_ref):
    _split(jnp.dot(x_ref[...], w_ref[...], preferred_element_type=jnp.float32),
           o_ref)


def _fuse_body(p_ref, w_ref, o_ref):
    h = jnp.maximum(jnp.concatenate([p_ref[0], p_ref[1]], axis=1), 0.0)
    _split(jnp.dot(h, w_ref[...], preferred_element_type=jnp.float32), o_ref)


def _make_sc_scatter(nh, acc_n, dh, chunks):
    rpt = acc_n // NUM_SUBCORES
    mesh = plsc.VectorSubcoreMesh(
        core_axis_name="c", subcore_axis_name="s",
        num_cores=NUM_CORES, num_subcores=NUM_SUBCORES)

    def body(h_hbm, src_hbm, dst_hbm, init_hbm, out_hbm,
             src_v, dst_v, rows0, rows1, acc, sem0, sem1):
        cid = lax.axis_index("c")
        sid = lax.axis_index("s")
        hc = h_hbm.at[cid]

        pltpu.sync_copy(src_hbm.at[sid], src_v)
        pltpu.sync_copy(dst_hbm.at[sid], dst_v)
        pltpu.sync_copy(init_hbm.at[cid].at[pl.ds(sid * rpt, rpt)],
                        acc.at[pl.ds(sid * rpt, rpt)])
        plsc.subcore_barrier()

        def gather(j, buf, sem):
            pltpu.async_copy(hc.at[src_v.at[j]], buf, sem)

        def gwait(j, buf, sem):
            pltpu.make_async_copy(hc.at[src_v.at[j]], buf, sem).wait()

        def scatter(j, buf):
            pltpu.sync_copy(buf, acc.at[dst_v.at[j]], add=True)

        gather(0, rows0, sem0)

        def step(i, carry):
            j = 2 * i
            gather(j + 1, rows1, sem1)
            gwait(j, rows0, sem0)
            scatter(j, rows0)
            gather(j + 2, rows0, sem0)
            gwait(j + 1, rows1, sem1)
            scatter(j + 1, rows1)
            return carry

        lax.fori_loop(0, chunks // 2 - 1, step, 0)
        j = chunks - 2
        gather(j + 1, rows1, sem1)
        gwait(j, rows0, sem0)
        scatter(j, rows0)
        gwait(j + 1, rows1, sem1)
        scatter(j + 1, rows1)

        plsc.subcore_barrier()
        pltpu.sync_copy(acc.at[pl.ds(sid * rpt, rpt)],
                        out_hbm.at[cid].at[pl.ds(sid * rpt, rpt)])

    return pl.kernel(
        body,
        out_type=jax.ShapeDtypeStruct((NUM_CORES, acc_n, dh), jnp.float32),
        mesh=mesh,
        compiler_params=pltpu.CompilerParams(use_tc_tiling_on_sc=False),
        scratch_types=[
            pltpu.VMEM((chunks, C), jnp.int32),
            pltpu.VMEM((chunks, C), jnp.int32),
            pltpu.VMEM((C, dh), jnp.float32),
            pltpu.VMEM((C, dh), jnp.float32),
            pltpu.VMEM_SHARED((acc_n, dh), jnp.float32),
            pltpu.SemaphoreType.DMA,
            pltpu.SemaphoreType.DMA,
        ],
    )


def kernel(x, edge_index, W1, b1, W2, b2):
    n, d_in = x.shape
    d_hid = W1.shape[1]
    d_out = W2.shape[1]
    e = edge_index.shape[1]
    hh = d_hid // 2
    ho = d_out // 2

    chunks = -(-e // (NUM_SUBCORES * C))
    chunks += chunks % 2
    e_pad = NUM_SUBCORES * chunks * C
    acc_n = (-(-(n + 1) // (NUM_SUBCORES * 8))) * NUM_SUBCORES * 8
    dummy = n
    src = jnp.concatenate(
        [edge_index[0], jnp.zeros((e_pad - e,), jnp.int32)]
    ).reshape(NUM_SUBCORES, chunks, C)
    dst = jnp.concatenate(
        [edge_index[1], jnp.full((e_pad - e,), dummy, jnp.int32)]
    ).reshape(NUM_SUBCORES, chunks, C)
    init1 = jnp.broadcast_to(b1.reshape(2, 1, hh), (2, acc_n, hh))
    init2 = jnp.broadcast_to(b2.reshape(2, 1, ho), (2, acc_n, ho))

    h1 = pl.pallas_call(
        _mm_body,
        out_shape=jax.ShapeDtypeStruct((2, n, hh), jnp.float32))(x, W1)
    p1 = _make_sc_scatter(n, acc_n, hh, chunks)(h1, src, dst, init1)
    h2 = pl.pallas_call(
        _fuse_body,
        out_shape=jax.ShapeDtypeStruct((2, acc_n, ho), jnp.float32))(p1, W2)
    p2 = _make_sc_scatter(acc_n, acc_n, ho, chunks)(h2, src, dst, init2)
    return jnp.concatenate([p2[0, :n], p2[1, :n]], axis=1)
